# baseline (speedup 1.0000x reference)
import functools
import math

import jax
import jax.numpy as jnp
from jax import lax
from jax.experimental import pallas as pl
from jax.experimental.pallas import tpu as pltpu

_SMOOTHING = 0.1
_PAD_IDX = 0

_R = 128
_NBUF = 8


def _copy(pred_hbm, tgt_hbm, bufs, tbufs, sems, tsems, g, b):
    pc = pltpu.make_async_copy(
        pred_hbm.at[pl.ds(g * _R, _R), :], bufs.at[b], sems.at[b])
    tc = pltpu.make_async_copy(
        tgt_hbm.at[pl.ds(g * _R, _R), :], tbufs.at[b], tsems.at[b])
    return pc, tc


def _body(tgt_hbm, reward_ref, pred_hbm, out_ref, bufs, tbufs, acc_ref,
          sems, tsems, *, nsteps, V):
    u = _SMOOTHING / (V - 2)

    for b in range(4):
        pc, tc = _copy(pred_hbm, tgt_hbm, bufs, tbufs, sems, tsems, b, b)
        pc.start()
        tc.start()

    acc_ref[0] = 0.0
    acc_ref[1] = 0.0

    def cycle(it, _):
        for b in range(_NBUF):
            g = it * _NBUF + b
            gpre = g + 4
            bpre = (b + 4) % _NBUF

            @pl.when(gpre < nsteps)
            def _pre():
                pc, tc = _copy(pred_hbm, tgt_hbm, bufs, tbufs, sems, tsems,
                               gpre, bpre)
                pc.start()
                tc.start()

            pc, tc = _copy(pred_hbm, tgt_hbm, bufs, tbufs, sems, tsems, g, b)
            pc.wait()
            tc.wait()

            p = bufs[b]
            t2 = tbufs[b]
            valid2 = t2 != _PAD_IDX
            col = lax.broadcasted_iota(jnp.int32, (_R, V), 1)
            is_t = col == t2
            pt2 = jnp.sum(jnp.where(is_t, p, 0.0), axis=1, keepdims=True)
            rowsum2 = jnp.sum(p, axis=1, keepdims=True)
            p02 = p[:, 0:1]
            row_dp = u * (rowsum2 - p02 - pt2) + (1.0 - _SMOOTHING) * pt2
            acc_ref[0] += jnp.sum(jnp.where(valid2, row_dp, 0.0))
            acc_ref[1] += jnp.sum(valid2.astype(jnp.float32))
        return 0

    lax.fori_loop(0, nsteps // _NBUF, cycle, 0)

    C = (V - 2) * u * math.log(u) + (1.0 - _SMOOTHING) * math.log(1.0 - _SMOOTHING)
    total = acc_ref[1] * C - acc_ref[0]
    out_ref[0] = total / (nsteps * _R * V) * reward_ref[0]


def kernel(pred, target, reward):
    B, S, V = pred.shape
    N = B * S
    pred2 = pred.reshape(N, V)
    tgt = target.reshape(N, 1).astype(jnp.int32)
    nsteps = N // _R

    out = pl.pallas_call(
        functools.partial(_body, nsteps=nsteps, V=V),
        in_specs=[
            pl.BlockSpec(memory_space=pltpu.MemorySpace.HBM),
            pl.BlockSpec(memory_space=pltpu.MemorySpace.SMEM),
            pl.BlockSpec(memory_space=pltpu.MemorySpace.HBM),
        ],
        out_specs=pl.BlockSpec(memory_space=pltpu.MemorySpace.SMEM),
        out_shape=jax.ShapeDtypeStruct((1,), jnp.float32),
        scratch_shapes=[
            pltpu.VMEM((_NBUF, _R, V), jnp.float32),
            pltpu.VMEM((_NBUF, _R, 1), jnp.int32),
            pltpu.SMEM((2,), jnp.float32),
            pltpu.SemaphoreType.DMA((_NBUF,)),
            pltpu.SemaphoreType.DMA((_NBUF,)),
        ],
    )(tgt, reward, pred2)
    return out


# speedup vs baseline: 1.0005x; 1.0005x over previous
import functools
import math

import jax
import jax.numpy as jnp
from jax import lax
from jax.experimental import pallas as pl
from jax.experimental.pallas import tpu as pltpu

_SMOOTHING = 0.1
_PAD_IDX = 0

_R = 128
_NBUF = 8


def _copy(pred_hbm, tgt_hbm, bufs, tbufs, sems, tsems, g, b):
    pc = pltpu.make_async_copy(
        pred_hbm.at[pl.ds(g * _R, _R), :], bufs.at[b], sems.at[b])
    tc = pltpu.make_async_copy(
        tgt_hbm.at[pl.ds(g * _R, _R), :], tbufs.at[b], tsems.at[b])
    return pc, tc


def _body(tgt_hbm, reward_ref, pred_hbm, out_ref, bufs, tbufs, acc_ref,
          sems, tsems, *, nsteps, V):
    u = _SMOOTHING / (V - 2)

    for b in range(3):
        pc, tc = _copy(pred_hbm, tgt_hbm, bufs, tbufs, sems, tsems, b, b)
        pc.start()
        tc.start()

    acc_ref[0] = 0.0
    acc_ref[1] = 0.0

    def cycle(it, _):
        for b in range(_NBUF):
            g = it * _NBUF + b
            gpre = g + 3
            bpre = (b + 3) % _NBUF

            @pl.when(gpre < nsteps)
            def _pre():
                pc, tc = _copy(pred_hbm, tgt_hbm, bufs, tbufs, sems, tsems,
                               gpre, bpre)
                pc.start()
                tc.start()

            pc, tc = _copy(pred_hbm, tgt_hbm, bufs, tbufs, sems, tsems, g, b)
            pc.wait()
            tc.wait()

            p = bufs[b]
            t2 = tbufs[b]
            valid2 = t2 != _PAD_IDX
            col = lax.broadcasted_iota(jnp.int32, (_R, V), 1)
            is_t = col == t2
            pt2 = jnp.sum(jnp.where(is_t, p, 0.0), axis=1, keepdims=True)
            rowsum2 = jnp.sum(p, axis=1, keepdims=True)
            p02 = p[:, 0:1]
            row_dp = u * (rowsum2 - p02 - pt2) + (1.0 - _SMOOTHING) * pt2
            acc_ref[0] += jnp.sum(jnp.where(valid2, row_dp, 0.0))
            acc_ref[1] += jnp.sum(valid2.astype(jnp.float32))
        return 0

    lax.fori_loop(0, nsteps // _NBUF, cycle, 0)

    C = (V - 2) * u * math.log(u) + (1.0 - _SMOOTHING) * math.log(1.0 - _SMOOTHING)
    total = acc_ref[1] * C - acc_ref[0]
    out_ref[0] = total / (nsteps * _R * V) * reward_ref[0]


def kernel(pred, target, reward):
    B, S, V = pred.shape
    N = B * S
    pred2 = pred.reshape(N, V)
    tgt = target.reshape(N, 1).astype(jnp.int32)
    nsteps = N // _R

    out = pl.pallas_call(
        functools.partial(_body, nsteps=nsteps, V=V),
        in_specs=[
            pl.BlockSpec(memory_space=pltpu.MemorySpace.HBM),
            pl.BlockSpec(memory_space=pltpu.MemorySpace.SMEM),
            pl.BlockSpec(memory_space=pltpu.MemorySpace.HBM),
        ],
        out_specs=pl.BlockSpec(memory_space=pltpu.MemorySpace.SMEM),
        out_shape=jax.ShapeDtypeStruct((1,), jnp.float32),
        scratch_shapes=[
            pltpu.VMEM((_NBUF, _R, V), jnp.float32),
            pltpu.VMEM((_NBUF, _R, 1), jnp.int32),
            pltpu.SMEM((2,), jnp.float32),
            pltpu.SemaphoreType.DMA((_NBUF,)),
            pltpu.SemaphoreType.DMA((_NBUF,)),
        ],
    )(tgt, reward, pred2)
    return out
